# Initial kernel scaffold; baseline (speedup 1.0000x reference)
import jax
import jax.numpy as jnp
from jax.experimental import pallas as pl


def kernel(x, pos_emb_weight):
    raise NotImplementedError("write your pallas kernel here")


# speedup vs baseline: 3.6021x; 3.6021x over previous
import functools

import jax
import jax.numpy as jnp
from jax import lax
from jax.experimental import pallas as pl
from jax.experimental.pallas import tpu as pltpu
from jax.experimental.pallas import tpu_sc as plsc

NPOS = 1000
EMB_DIM = 64
BATCH = 4096
HIST = 200

NC = 2
NS = 16
NW = NC * NS

B = BATCH * HIST
B_PER_W = B // NW

IDXW = 128
CHUNK_ROWS = 4
CHUNK = CHUNK_ROWS * IDXW
N_CHUNKS = B_PER_W // CHUNK
ROWS_PER_W = B_PER_W // IDXW

_mesh = plsc.VectorSubcoreMesh(core_axis_name="c", subcore_axis_name="s")


@functools.partial(
    pl.kernel,
    out_type=jax.ShapeDtypeStruct((B, EMB_DIM), jnp.float32),
    mesh=_mesh,
    compiler_params=pltpu.CompilerParams(use_tc_tiling_on_sc=False),
    scratch_types=[
        pltpu.VMEM((2, CHUNK_ROWS, IDXW), jnp.int32),
        pltpu.VMEM((2, CHUNK, EMB_DIM), jnp.float32),
        pltpu.SemaphoreType.DMA,
        pltpu.SemaphoreType.DMA,
        pltpu.SemaphoreType.DMA,
        pltpu.SemaphoreType.DMA,
        pltpu.SemaphoreType.DMA,
        pltpu.SemaphoreType.DMA,
    ],
)
def _emb_lookup(x_hbm, w_hbm, out_hbm, idx_v, rows_v,
                isem0, isem1, gsem0, gsem1, ssem0, ssem1):
    isem = (isem0, isem1)
    gsem = (gsem0, gsem1)
    ssem = (ssem0, ssem1)

    wid = lax.axis_index("s") * NC + lax.axis_index("c")
    base = wid * B_PER_W
    brow = wid * ROWS_PER_W

    def issue_idx_load(c, s):
        pltpu.async_copy(
            x_hbm.at[pl.ds(brow + c * CHUNK_ROWS, CHUNK_ROWS)],
            idx_v.at[s], isem[s])

    def wait_idx_load(c, s):
        pltpu.make_async_copy(
            x_hbm.at[pl.ds(brow + c * CHUNK_ROWS, CHUNK_ROWS)],
            idx_v.at[s], isem[s]).wait()

    def issue_store(c, s):
        pltpu.async_copy(
            rows_v.at[s], out_hbm.at[pl.ds(base + c * CHUNK, CHUNK)],
            ssem[s])

    def wait_store(c, s):
        pltpu.make_async_copy(
            rows_v.at[s], out_hbm.at[pl.ds(base + c * CHUNK, CHUNK)],
            ssem[s]).wait()

    issue_idx_load(0, 0)
    issue_idx_load(1, 1)

    @pl.loop(0, N_CHUNKS, step=2)
    def _(g):
        for s in range(2):
            c = g + s
            wait_idx_load(c, s)

            @pl.when(c >= 2)
            def _():
                wait_store(c - 2, s)

            gathers = []
            for j in range(CHUNK_ROWS):
                gathers.append(pltpu.async_copy(
                    w_hbm.at[idx_v.at[s, j]],
                    rows_v.at[s, pl.ds(j * IDXW, IDXW)],
                    gsem[s]))
            for g_ in gathers:
                g_.wait()

            @pl.when(c + 2 < N_CHUNKS)
            def _():
                issue_idx_load(c + 2, s)

            issue_store(c, s)

    wait_store(N_CHUNKS - 2, 0)
    wait_store(N_CHUNKS - 1, 1)


def kernel(x, pos_emb_weight):
    w = pos_emb_weight.at[0].set(0.0)
    x_rows = x.astype(jnp.int32).reshape(B // IDXW, IDXW)
    out = _emb_lookup(x_rows, w)
    return out.reshape(BATCH, HIST, EMB_DIM)


# speedup vs baseline: 5.0165x; 1.3926x over previous
import functools

import jax
import jax.numpy as jnp
from jax import lax
from jax.experimental import pallas as pl
from jax.experimental.pallas import tpu as pltpu
from jax.experimental.pallas import tpu_sc as plsc

NPOS = 1000
EMB_DIM = 64
BATCH = 4096
HIST = 200

NC = 2
NS = 16
NW = NC * NS

B = BATCH * HIST
B_PER_W = B // NW

IDXW = 128
CHUNK_ROWS = 4
CHUNK = CHUNK_ROWS * IDXW
N_CHUNKS = B_PER_W // CHUNK
ROWS_PER_W = B_PER_W // IDXW

_mesh = plsc.VectorSubcoreMesh(core_axis_name="c", subcore_axis_name="s")


@functools.partial(
    pl.kernel,
    out_type=jax.ShapeDtypeStruct((B, EMB_DIM), jnp.float32),
    mesh=_mesh,
    compiler_params=pltpu.CompilerParams(use_tc_tiling_on_sc=False),
    scratch_types=[
        pltpu.VMEM((2, CHUNK_ROWS, IDXW), jnp.int32),
        pltpu.VMEM((2, CHUNK, EMB_DIM), jnp.float32),
        pltpu.VMEM_SHARED((NPOS, EMB_DIM), jnp.float32),
        pltpu.SemaphoreType.DMA,
        pltpu.SemaphoreType.DMA,
        pltpu.SemaphoreType.DMA,
        pltpu.SemaphoreType.DMA,
        pltpu.SemaphoreType.DMA,
        pltpu.SemaphoreType.DMA,
    ],
)
def _emb_lookup(x_hbm, w_hbm, out_hbm, idx_v, rows_v, tab_sh,
                isem0, isem1, gsem0, gsem1, ssem0, ssem1):
    isem = (isem0, isem1)
    gsem = (gsem0, gsem1)
    ssem = (ssem0, ssem1)

    @pl.when(lax.axis_index("s") == 0)
    def _():
        pltpu.sync_copy(w_hbm, tab_sh)

    plsc.subcore_barrier()

    wid = lax.axis_index("s") * NC + lax.axis_index("c")
    base = wid * B_PER_W
    brow = wid * ROWS_PER_W

    def issue_idx_load(c, s):
        pltpu.async_copy(
            x_hbm.at[pl.ds(brow + c * CHUNK_ROWS, CHUNK_ROWS)],
            idx_v.at[s], isem[s])

    def wait_idx_load(c, s):
        pltpu.make_async_copy(
            x_hbm.at[pl.ds(brow + c * CHUNK_ROWS, CHUNK_ROWS)],
            idx_v.at[s], isem[s]).wait()

    def issue_store(c, s):
        pltpu.async_copy(
            rows_v.at[s], out_hbm.at[pl.ds(base + c * CHUNK, CHUNK)],
            ssem[s])

    def wait_store(c, s):
        pltpu.make_async_copy(
            rows_v.at[s], out_hbm.at[pl.ds(base + c * CHUNK, CHUNK)],
            ssem[s]).wait()

    issue_idx_load(0, 0)
    issue_idx_load(1, 1)

    @pl.loop(0, N_CHUNKS, step=2)
    def _(g):
        for s in range(2):
            c = g + s
            wait_idx_load(c, s)

            @pl.when(c >= 2)
            def _():
                wait_store(c - 2, s)

            gathers = []
            for j in range(CHUNK_ROWS):
                gathers.append(pltpu.async_copy(
                    tab_sh.at[idx_v.at[s, j]],
                    rows_v.at[s, pl.ds(j * IDXW, IDXW)],
                    gsem[s]))
            for g_ in gathers:
                g_.wait()

            @pl.when(c + 2 < N_CHUNKS)
            def _():
                issue_idx_load(c + 2, s)

            issue_store(c, s)

    wait_store(N_CHUNKS - 2, 0)
    wait_store(N_CHUNKS - 1, 1)


def kernel(x, pos_emb_weight):
    w = pos_emb_weight.at[0].set(0.0)
    x_rows = x.astype(jnp.int32).reshape(B // IDXW, IDXW)
    out = _emb_lookup(x_rows, w)
    return out.reshape(BATCH, HIST, EMB_DIM)


# speedup vs baseline: 5.0323x; 1.0032x over previous
import functools

import jax
import jax.numpy as jnp
from jax import lax
from jax.experimental import pallas as pl
from jax.experimental.pallas import tpu as pltpu
from jax.experimental.pallas import tpu_sc as plsc

NPOS = 1000
EMB_DIM = 64
BATCH = 4096
HIST = 200

NC = 2
NS = 16
NW = NC * NS

B = BATCH * HIST
B_PER_W = B // NW

IDXW = 128
CHUNK_ROWS = 2
CHUNK = CHUNK_ROWS * IDXW
N_CHUNKS = B_PER_W // CHUNK
ROWS_PER_W = B_PER_W // IDXW
NBUF = 4

_mesh = plsc.VectorSubcoreMesh(core_axis_name="c", subcore_axis_name="s")


@functools.partial(
    pl.kernel,
    out_type=jax.ShapeDtypeStruct((B, EMB_DIM), jnp.float32),
    mesh=_mesh,
    compiler_params=pltpu.CompilerParams(use_tc_tiling_on_sc=False),
    scratch_types=[
        pltpu.VMEM((ROWS_PER_W, IDXW), jnp.int32),
        pltpu.VMEM((NBUF, CHUNK, EMB_DIM), jnp.float32),
        pltpu.VMEM_SHARED((NPOS, EMB_DIM), jnp.float32),
        pltpu.SemaphoreType.DMA,
        pltpu.SemaphoreType.DMA,
        pltpu.SemaphoreType.DMA,
        pltpu.SemaphoreType.DMA,
        pltpu.SemaphoreType.DMA,
        pltpu.SemaphoreType.DMA,
        pltpu.SemaphoreType.DMA,
        pltpu.SemaphoreType.DMA,
        pltpu.SemaphoreType.DMA,
    ],
)
def _emb_lookup(x_hbm, w_hbm, out_hbm, idx_all, rows_v, tab_sh,
                gsem0, gsem1, gsem2, gsem3, ssem0, ssem1, ssem2, ssem3,
                lsem):
    gsem = (gsem0, gsem1, gsem2, gsem3)
    ssem = (ssem0, ssem1, ssem2, ssem3)

    wid = lax.axis_index("s") * NC + lax.axis_index("c")
    base = wid * B_PER_W
    brow = wid * ROWS_PER_W

    idx_cp = pltpu.async_copy(
        x_hbm.at[pl.ds(brow, ROWS_PER_W)], idx_all, lsem)

    @pl.when(lax.axis_index("s") == 0)
    def _():
        pltpu.sync_copy(w_hbm, tab_sh)

    idx_cp.wait()
    plsc.subcore_barrier()

    def issue_gathers(c, s):
        for j in range(CHUNK_ROWS):
            pltpu.async_copy(
                tab_sh.at[idx_all.at[c * CHUNK_ROWS + j]],
                rows_v.at[s, pl.ds(j * IDXW, IDXW)],
                gsem[s])

    def wait_gathers(c, s):
        for j in range(CHUNK_ROWS):
            pltpu.make_async_copy(
                tab_sh.at[idx_all.at[c * CHUNK_ROWS + j]],
                rows_v.at[s, pl.ds(j * IDXW, IDXW)],
                gsem[s]).wait()

    def issue_store(c, s):
        pltpu.async_copy(
            rows_v.at[s], out_hbm.at[pl.ds(base + c * CHUNK, CHUNK)],
            ssem[s])

    def wait_store(c, s):
        pltpu.make_async_copy(
            rows_v.at[s], out_hbm.at[pl.ds(base + c * CHUNK, CHUNK)],
            ssem[s]).wait()

    for s in range(NBUF):
        issue_gathers(s, s)

    @pl.loop(0, N_CHUNKS, step=NBUF)
    def _(g):
        for s in range(NBUF):
            c = g + s
            wait_gathers(c, s)
            issue_store(c, s)

            @pl.when(c + NBUF < N_CHUNKS)
            def _():
                wait_store(c, s)
                issue_gathers(c + NBUF, s)

    for s in range(NBUF):
        wait_store(N_CHUNKS - NBUF + s, s)


def kernel(x, pos_emb_weight):
    w = pos_emb_weight.at[0].set(0.0)
    x_rows = x.astype(jnp.int32).reshape(B // IDXW, IDXW)
    out = _emb_lookup(x_rows, w)
    return out.reshape(BATCH, HIST, EMB_DIM)


# speedup vs baseline: 10.4787x; 2.0823x over previous
import functools

import jax
import jax.numpy as jnp
from jax import lax
from jax.experimental import pallas as pl
from jax.experimental.pallas import tpu as pltpu
from jax.experimental.pallas import tpu_sc as plsc

NPOS = 1000
EMB_DIM = 64
BATCH = 4096
HIST = 200

NC = 2
NS = 16
NW = NC * NS

B = BATCH * HIST
B_PER_W = B // NW

IDXW = 128
CHUNK_ROWS = 2
CHUNK = CHUNK_ROWS * IDXW
N_CHUNKS = B_PER_W // CHUNK
ROWS_PER_W = B_PER_W // IDXW
NBUF = 4

_mesh = plsc.VectorSubcoreMesh(core_axis_name="c", subcore_axis_name="s")


@functools.partial(
    pl.kernel,
    out_type=jax.ShapeDtypeStruct((B, 2 * EMB_DIM), jnp.float32),
    mesh=_mesh,
    compiler_params=pltpu.CompilerParams(use_tc_tiling_on_sc=False),
    scratch_types=[
        pltpu.VMEM((ROWS_PER_W, IDXW), jnp.int32),
        pltpu.VMEM((NBUF, CHUNK, EMB_DIM), jnp.float32),
        pltpu.VMEM_SHARED((NPOS, EMB_DIM), jnp.float32),
        pltpu.SemaphoreType.DMA,
        pltpu.SemaphoreType.DMA,
        pltpu.SemaphoreType.DMA,
        pltpu.SemaphoreType.DMA,
        pltpu.SemaphoreType.DMA,
        pltpu.SemaphoreType.DMA,
        pltpu.SemaphoreType.DMA,
        pltpu.SemaphoreType.DMA,
        pltpu.SemaphoreType.DMA,
    ],
)
def _emb_lookup(x_hbm, w_hbm, out_hbm, idx_all, rows_v, tab_sh,
                gsem0, gsem1, gsem2, gsem3, ssem0, ssem1, ssem2, ssem3,
                lsem):
    gsem = (gsem0, gsem1, gsem2, gsem3)
    ssem = (ssem0, ssem1, ssem2, ssem3)

    wid = lax.axis_index("s") * NC + lax.axis_index("c")
    base = wid * B_PER_W
    brow = wid * ROWS_PER_W

    idx_cp = pltpu.async_copy(
        x_hbm.at[pl.ds(brow, ROWS_PER_W)], idx_all, lsem)

    @pl.when(lax.axis_index("s") == 0)
    def _():
        pltpu.sync_copy(w_hbm, tab_sh)

    idx_cp.wait()
    plsc.subcore_barrier()

    def issue_gathers(c, s):
        for j in range(CHUNK_ROWS):
            pltpu.async_copy(
                tab_sh.at[idx_all.at[c * CHUNK_ROWS + j]],
                rows_v.at[s, pl.ds(j * IDXW, IDXW)],
                gsem[s])

    def wait_gathers(c, s):
        for j in range(CHUNK_ROWS):
            pltpu.make_async_copy(
                tab_sh.at[idx_all.at[c * CHUNK_ROWS + j]],
                rows_v.at[s, pl.ds(j * IDXW, IDXW)],
                gsem[s]).wait()

    def issue_store(c, s):
        pltpu.async_copy(
            rows_v.at[s],
            out_hbm.at[pl.ds(base + c * CHUNK, CHUNK), pl.ds(0, EMB_DIM)],
            ssem[s])

    def wait_store(c, s):
        pltpu.make_async_copy(
            rows_v.at[s],
            out_hbm.at[pl.ds(base + c * CHUNK, CHUNK), pl.ds(0, EMB_DIM)],
            ssem[s]).wait()

    for s in range(NBUF):
        issue_gathers(s, s)

    @pl.loop(0, N_CHUNKS, step=NBUF)
    def _(g):
        for s in range(NBUF):
            c = g + s
            wait_gathers(c, s)
            issue_store(c, s)

            @pl.when(c + NBUF < N_CHUNKS)
            def _():
                wait_store(c, s)
                issue_gathers(c + NBUF, s)

    for s in range(NBUF):
        wait_store(N_CHUNKS - NBUF + s, s)


def kernel(x, pos_emb_weight):
    w = pos_emb_weight.at[0].set(0.0)
    x_rows = x.astype(jnp.int32).reshape(B // IDXW, IDXW)
    out = _emb_lookup(x_rows, w)
    return out[:, :EMB_DIM].reshape(BATCH, HIST, EMB_DIM)


# speedup vs baseline: 10.5266x; 1.0046x over previous
import functools

import jax
import jax.numpy as jnp
from jax import lax
from jax.experimental import pallas as pl
from jax.experimental.pallas import tpu as pltpu
from jax.experimental.pallas import tpu_sc as plsc

NPOS = 1000
EMB_DIM = 64
BATCH = 4096
HIST = 200

NC = 2
NS = 16
NW = NC * NS

B = BATCH * HIST
B_PER_W = B // NW

IDXW = 128
CHUNK_ROWS = 1
CHUNK = CHUNK_ROWS * IDXW
N_CHUNKS = B_PER_W // CHUNK
ROWS_PER_W = B_PER_W // IDXW
NBUF = 8

_mesh = plsc.VectorSubcoreMesh(core_axis_name="c", subcore_axis_name="s")


@functools.partial(
    pl.kernel,
    out_type=jax.ShapeDtypeStruct((B, 2 * EMB_DIM), jnp.float32),
    mesh=_mesh,
    compiler_params=pltpu.CompilerParams(use_tc_tiling_on_sc=False),
    scratch_types=[
        pltpu.VMEM((ROWS_PER_W, IDXW), jnp.int32),
        pltpu.VMEM((NBUF, CHUNK, EMB_DIM), jnp.float32),
        pltpu.VMEM_SHARED((NPOS, EMB_DIM), jnp.float32),
    ] + [pltpu.SemaphoreType.DMA] * 17,
)
def _emb_lookup(x_hbm, w_hbm, out_hbm, idx_all, rows_v, tab_sh, *sems):
    gsem = sems[:NBUF]
    ssem = sems[NBUF:2 * NBUF]
    lsem = sems[2 * NBUF]

    wid = lax.axis_index("s") * NC + lax.axis_index("c")
    base = wid * B_PER_W
    brow = wid * ROWS_PER_W

    idx_cp = pltpu.async_copy(
        x_hbm.at[pl.ds(brow, ROWS_PER_W)], idx_all, lsem)

    @pl.when(lax.axis_index("s") == 0)
    def _():
        pltpu.sync_copy(w_hbm, tab_sh)

    idx_cp.wait()
    plsc.subcore_barrier()

    def issue_gathers(c, s):
        for j in range(CHUNK_ROWS):
            pltpu.async_copy(
                tab_sh.at[idx_all.at[c * CHUNK_ROWS + j]],
                rows_v.at[s, pl.ds(j * IDXW, IDXW)],
                gsem[s])

    def wait_gathers(c, s):
        for j in range(CHUNK_ROWS):
            pltpu.make_async_copy(
                tab_sh.at[idx_all.at[c * CHUNK_ROWS + j]],
                rows_v.at[s, pl.ds(j * IDXW, IDXW)],
                gsem[s]).wait()

    def issue_store(c, s):
        pltpu.async_copy(
            rows_v.at[s],
            out_hbm.at[pl.ds(base + c * CHUNK, CHUNK), pl.ds(0, EMB_DIM)],
            ssem[s])

    def wait_store(c, s):
        pltpu.make_async_copy(
            rows_v.at[s],
            out_hbm.at[pl.ds(base + c * CHUNK, CHUNK), pl.ds(0, EMB_DIM)],
            ssem[s]).wait()

    for s in range(NBUF):
        issue_gathers(s, s)

    @pl.loop(0, N_CHUNKS, step=NBUF)
    def _(g):
        for s in range(NBUF):
            c = g + s
            wait_gathers(c, s)
            issue_store(c, s)

            @pl.when(c + NBUF < N_CHUNKS)
            def _():
                wait_store(c, s)
                issue_gathers(c + NBUF, s)

    for s in range(NBUF):
        wait_store(N_CHUNKS - NBUF + s, s)


def kernel(x, pos_emb_weight):
    w = pos_emb_weight.at[0].set(0.0)
    x_rows = x.astype(jnp.int32).reshape(B // IDXW, IDXW)
    out = _emb_lookup(x_rows, w)
    return out[:, :EMB_DIM].reshape(BATCH, HIST, EMB_DIM)
